# baseline (speedup 1.0000x reference)
import functools

import jax
import jax.numpy as jnp
from jax import lax
from jax.experimental import pallas as pl
from jax.experimental.pallas import tpu as pltpu
from jax.experimental.pallas import tpu_sc as plsc

NC = 2
NS = 16
NW = NC * NS
LRELU = 0.1
BN_EPS = 1e-5


def _make_sc_spmm(N, D, E, with_deg):
    CH = 256
    npc = E // (NW * CH)
    assert E == NW * npc * CH and npc % 2 == 0
    NP = NS * 640

    mesh = plsc.VectorSubcoreMesh(
        core_axis_name="c", subcore_axis_name="s",
        num_cores=NC, num_subcores=NS)

    out_type = [jax.ShapeDtypeStruct((NC, N, D), jnp.float32)]
    if with_deg:
        out_type.append(jax.ShapeDtypeStruct((NC, NP), jnp.float32))

    scratch = [
        pltpu.VMEM_SHARED((N + 16, D), jnp.float32),
        pltpu.VMEM_SHARED((NP,), jnp.float32),
        pltpu.VMEM((CH,), jnp.int32),
        pltpu.VMEM((CH,), jnp.int32),
        pltpu.VMEM((CH,), jnp.int32),
        pltpu.VMEM((CH,), jnp.int32),
        pltpu.VMEM((CH // 8, 128), jnp.float32),
        pltpu.VMEM((CH // 8, 128), jnp.float32),
        pltpu.VMEM((CH, D), jnp.float32),
        pltpu.VMEM((CH,), jnp.float32),
        pltpu.VMEM((640,), jnp.float32),
        pltpu.SemaphoreType.DMA,
        pltpu.SemaphoreType.DMA,
        pltpu.SemaphoreType.DMA,
    ]

    @functools.partial(pl.kernel, out_type=tuple(out_type), mesh=mesh,
                       scratch_types=scratch)
    def spmm(*refs):
        if with_deg:
            (h_hbm, src_hbm, dst_hbm, w_hbm, z2_hbm,
             out_hbm, deg_hbm, acc, deg_sh,
             src0, src1, dst0, dst1, w0, w1, rows_v, ones_v, zer_v,
             sem_m0, sem_m1, sem) = refs
        else:
            (h_hbm, src_hbm, dst_hbm, w_hbm, z2_hbm,
             out_hbm, acc, deg_sh,
             src0, src1, dst0, dst1, w0, w1, rows_v, ones_v, zer_v,
             sem_m0, sem_m1, sem) = refs
            deg_hbm = None
        src_v = (src0, src1)
        dst_v = (dst0, dst1)
        w_v = (w0, w1)
        sem_m = (sem_m0, sem_m1)

        c = lax.axis_index("c")
        s = lax.axis_index("s")
        wid = s * NC + c

        @pl.when(s < NS - 1)
        def _():
            pltpu.sync_copy(z2_hbm.at[pl.ds(s * 640, 640)],
                            acc.at[pl.ds(s * 640, 640)])

        @pl.when(s == NS - 1)
        def _():
            pltpu.sync_copy(z2_hbm.at[pl.ds(9600, 400)],
                            acc.at[pl.ds(9600, 400)])

        if with_deg:
            for i in range(640 // 16):
                zer_v[pl.ds(i * 16, 16)] = jnp.zeros((16,), jnp.float32)
            pltpu.sync_copy(zer_v, deg_sh.at[pl.ds(s * 640, 640)])
            for i in range(CH // 16):
                ones_v[pl.ds(i * 16, 16)] = jnp.ones((16,), jnp.float32)

        plsc.subcore_barrier()

        c0 = wid * npc
        clast = c0 + npc - 1

        def meta_start(ci, p):
            ci = jnp.minimum(ci, clast)
            base = pl.multiple_of(ci * CH, 256)
            pltpu.async_copy(src_hbm.at[pl.ds(base, CH)], src_v[p], sem_m[p])
            pltpu.async_copy(dst_hbm.at[pl.ds(base, CH)], dst_v[p], sem_m[p])
            pltpu.async_copy(
                w_hbm.at[pl.ds(pl.multiple_of(base // 8, 8), CH // 8)],
                w_v[p], sem_m[p])

        def meta_wait(p):
            pltpu.make_async_copy(src_hbm.at[pl.ds(0, CH)], src_v[p],
                                  sem_m[p]).wait()
            pltpu.make_async_copy(dst_hbm.at[pl.ds(0, CH)], dst_v[p],
                                  sem_m[p]).wait()
            pltpu.make_async_copy(w_hbm.at[pl.ds(0, CH // 8)], w_v[p],
                                  sem_m[p]).wait()

        def process(a, p):
            meta_wait(p)
            pltpu.async_copy(h_hbm.at[src_v[p]], rows_v, sem).wait()

            def grp(g, carry2):
                for j in range(16):
                    e = g * 16 + j
                    bc = w_v[p][2 * g + (j // 8), pl.ds((j % 8) * 16, 16)]
                    for k in range(D // 16):
                        sl = pl.ds(k * 16, 16)
                        rows_v[e, sl] = rows_v[e, sl] * bc
                return carry2

            lax.fori_loop(0, CH // 16, grp, 0)

            pltpu.sync_copy(rows_v, acc.at[dst_v[p]], add=True)
            if with_deg:
                pltpu.sync_copy(ones_v, deg_sh.at[dst_v[p]], add=True)
            meta_start(a + 2, p)

        meta_start(c0, 0)
        meta_start(c0 + 1, 1)

        def chunk(t, carry):
            a = c0 + 2 * t
            process(a, 0)
            process(a + 1, 1)
            return carry

        lax.fori_loop(0, npc // 2, chunk, 0)
        meta_wait(0)
        meta_wait(1)

        plsc.subcore_barrier()

        @pl.when(s < NS - 1)
        def _():
            pltpu.sync_copy(acc.at[pl.ds(s * 640, 640)],
                            out_hbm.at[c, pl.ds(s * 640, 640)])

        @pl.when(s == NS - 1)
        def _():
            pltpu.sync_copy(acc.at[pl.ds(9600, 400)],
                            out_hbm.at[c, pl.ds(9600, 400)])

        if with_deg:
            pltpu.sync_copy(deg_sh.at[pl.ds(s * 640, 640)],
                            deg_hbm.at[c, pl.ds(s * 640, 640)])

    return spmm


def _dense_layer(p, deg3, h, Wl, bl, Wr, gamma, beta, final):
    N, D = h.shape
    R = 2000
    grid = (N // R,)

    def body(p_ref, deg_ref, h_ref, wl_ref, bl_ref, wr_ref, g_ref, b_ref,
             o_ref):
        pb = p_ref[0] + p_ref[1]
        degb = deg_ref[0, :, 0] + deg_ref[1, :, 0]
        inv = 1.0 / jnp.maximum(degb, 1.0)
        agg = pb * inv[:, None]
        hh = (jnp.dot(agg, wl_ref[...], preferred_element_type=jnp.float32)
              + bl_ref[...]
              + jnp.dot(h_ref[...], wr_ref[...],
                        preferred_element_type=jnp.float32))
        if final:
            nrm = jnp.sqrt(jnp.sum(hh * hh, axis=1, keepdims=True))
            o_ref[...] = hh / jnp.maximum(nrm, 1e-12)
        else:
            scale = g_ref[...] * (1.0 / jnp.sqrt(1.0 + BN_EPS))
            hh = hh * scale + b_ref[...]
            o_ref[...] = jnp.where(hh >= 0, hh, LRELU * hh)

    return pl.pallas_call(
        body,
        grid=grid,
        in_specs=[
            pl.BlockSpec((NC, R, D), lambda i: (0, i, 0)),
            pl.BlockSpec((NC, R, 1), lambda i: (0, i, 0)),
            pl.BlockSpec((R, D), lambda i: (i, 0)),
            pl.BlockSpec((D, D), lambda i: (0, 0)),
            pl.BlockSpec((1, D), lambda i: (0, 0)),
            pl.BlockSpec((D, D), lambda i: (0, 0)),
            pl.BlockSpec((1, D), lambda i: (0, 0)),
            pl.BlockSpec((1, D), lambda i: (0, 0)),
        ],
        out_specs=pl.BlockSpec((R, D), lambda i: (i, 0)),
        out_shape=jax.ShapeDtypeStruct((N, D), jnp.float32),
    )(p, deg3, h, Wl, bl.reshape(1, D), Wr, gamma.reshape(1, D),
      beta.reshape(1, D))


def kernel(x, edge_index, edge_weight, W1l, b1l, W1r, W2l, b2l, W2r,
           W3l, b3l, W3r, g1, be1, g2, be2):
    N, D = x.shape
    E = edge_weight.shape[0]
    CHT = NW * 256 * 2
    Ep = -(-E // CHT) * CHT
    pad = Ep - E
    src = jnp.concatenate([edge_index[0], jnp.zeros((pad,), jnp.int32)])
    tr = N + (jnp.arange(pad, dtype=jnp.int32) % 16)
    dst = jnp.concatenate([edge_index[1], tr])
    w = jnp.concatenate([edge_weight, jnp.zeros((pad,), jnp.float32)])
    w128 = jnp.repeat(w, 16).reshape(Ep // 8, 128)
    z2 = jnp.zeros((N, D), jnp.float32)

    spmm_deg = _make_sc_spmm(N, D, Ep, with_deg=True)
    spmm = _make_sc_spmm(N, D, Ep, with_deg=False)

    p1, deg = spmm_deg(x, src, dst, w128, z2)
    deg3 = deg[:, :N].reshape(NC, N, 1)
    h1 = _dense_layer(p1, deg3, x, W1l, b1l, W1r, g1, be1, final=False)
    (p2,) = spmm(h1, src, dst, w128, z2)
    h2 = _dense_layer(p2, deg3, h1, W2l, b2l, W2r, g2, be2, final=False)
    (p3,) = spmm(h2, src, dst, w128, z2)
    out = _dense_layer(p3, deg3, h2, W3l, b3l, W3r, g1, be1, final=True)
    return out


# speedup vs baseline: 1.7834x; 1.7834x over previous
import functools

import jax
import jax.numpy as jnp
from jax import lax
from jax.experimental import pallas as pl
from jax.experimental.pallas import tpu as pltpu
from jax.experimental.pallas import tpu_sc as plsc

NC = 2
NS = 16
NW = NC * NS
LRELU = 0.1
BN_EPS = 1e-5


def _make_sc_spmm(N, D, E, with_deg):
    CH = 256
    nfull = E // (NW * CH)
    nextra = E // CH - NW * nfull
    assert E == (NW * nfull + nextra) * CH
    NP = NS * 640

    mesh = plsc.VectorSubcoreMesh(
        core_axis_name="c", subcore_axis_name="s",
        num_cores=NC, num_subcores=NS)

    out_type = [jax.ShapeDtypeStruct((NC, N, D), jnp.float32)]
    if with_deg:
        out_type.append(jax.ShapeDtypeStruct((NC, NP), jnp.float32))

    scratch = [
        pltpu.VMEM_SHARED((N, D), jnp.float32),
        pltpu.VMEM_SHARED((NP,), jnp.float32),
        pltpu.VMEM((CH,), jnp.int32),
        pltpu.VMEM((CH,), jnp.int32),
        pltpu.VMEM((CH // 8, 128), jnp.float32),
        pltpu.VMEM((CH, D), jnp.float32),
        pltpu.VMEM((CH,), jnp.float32),
        pltpu.VMEM((640,), jnp.float32),
        pltpu.SemaphoreType.DMA,
    ]

    @functools.partial(pl.kernel, out_type=tuple(out_type), mesh=mesh,
                       scratch_types=scratch)
    def spmm(*refs):
        if with_deg:
            (h_hbm, src_hbm, dst_hbm, w_hbm, z2_hbm,
             out_hbm, deg_hbm, acc, deg_sh,
             src_v, dst_v, w_v, rows_v, ones_v, zer_v, sem) = refs
        else:
            (h_hbm, src_hbm, dst_hbm, w_hbm, z2_hbm,
             out_hbm, acc, deg_sh,
             src_v, dst_v, w_v, rows_v, ones_v, zer_v, sem) = refs
            deg_hbm = None

        c = lax.axis_index("c")
        s = lax.axis_index("s")
        wid = s * NC + c

        @pl.when(s < NS - 1)
        def _():
            pltpu.sync_copy(z2_hbm.at[pl.ds(s * 640, 640)],
                            acc.at[pl.ds(s * 640, 640)])

        @pl.when(s == NS - 1)
        def _():
            pltpu.sync_copy(z2_hbm.at[pl.ds(9600, 400)],
                            acc.at[pl.ds(9600, 400)])

        if with_deg:
            for i in range(640 // 16):
                zer_v[pl.ds(i * 16, 16)] = jnp.zeros((16,), jnp.float32)
            pltpu.sync_copy(zer_v, deg_sh.at[pl.ds(s * 640, 640)])
            for i in range(CH // 16):
                ones_v[pl.ds(i * 16, 16)] = jnp.ones((16,), jnp.float32)

        plsc.subcore_barrier()

        base0 = nfull * CH * wid + CH * jnp.minimum(wid, nextra)
        nch = nfull + (wid < nextra).astype(jnp.int32)

        def chunk(i, carry):
            base = pl.multiple_of(base0 + i * CH, 256)
            pltpu.sync_copy(src_hbm.at[pl.ds(base, CH)], src_v)
            pltpu.sync_copy(dst_hbm.at[pl.ds(base, CH)], dst_v)
            pltpu.sync_copy(
                w_hbm.at[pl.ds(pl.multiple_of(base // 8, 8), CH // 8)], w_v)
            pltpu.async_copy(h_hbm.at[src_v], rows_v, sem).wait()

            def grp(g, carry2):
                for j in range(16):
                    e = g * 16 + j
                    bc = w_v[2 * g + (j // 8), pl.ds((j % 8) * 16, 16)]
                    for k in range(D // 16):
                        sl = pl.ds(k * 16, 16)
                        rows_v[e, sl] = rows_v[e, sl] * bc
                return carry2

            lax.fori_loop(0, CH // 16, grp, 0)

            pltpu.sync_copy(rows_v, acc.at[dst_v], add=True)
            if with_deg:
                pltpu.sync_copy(ones_v, deg_sh.at[dst_v], add=True)
            return carry

        lax.fori_loop(0, nch, chunk, 0)

        plsc.subcore_barrier()

        @pl.when(s < NS - 1)
        def _():
            pltpu.sync_copy(acc.at[pl.ds(s * 640, 640)],
                            out_hbm.at[c, pl.ds(s * 640, 640)])

        @pl.when(s == NS - 1)
        def _():
            pltpu.sync_copy(acc.at[pl.ds(9600, 400)],
                            out_hbm.at[c, pl.ds(9600, 400)])

        if with_deg:
            pltpu.sync_copy(deg_sh.at[pl.ds(s * 640, 640)],
                            deg_hbm.at[c, pl.ds(s * 640, 640)])

    return spmm


def _dense_layer(p, deg3, h, Wl, bl, Wr, gamma, beta, final):
    N, D = h.shape
    R = 2000
    grid = (N // R,)

    def body(p_ref, deg_ref, h_ref, wl_ref, bl_ref, wr_ref, g_ref, b_ref,
             o_ref):
        pb = p_ref[0] + p_ref[1]
        degb = deg_ref[0, :, 0] + deg_ref[1, :, 0]
        inv = 1.0 / jnp.maximum(degb, 1.0)
        agg = pb * inv[:, None]
        hh = (jnp.dot(agg, wl_ref[...], preferred_element_type=jnp.float32)
              + bl_ref[...]
              + jnp.dot(h_ref[...], wr_ref[...],
                        preferred_element_type=jnp.float32))
        if final:
            nrm = jnp.sqrt(jnp.sum(hh * hh, axis=1, keepdims=True))
            o_ref[...] = hh / jnp.maximum(nrm, 1e-12)
        else:
            scale = g_ref[...] * (1.0 / jnp.sqrt(1.0 + BN_EPS))
            hh = hh * scale + b_ref[...]
            o_ref[...] = jnp.where(hh >= 0, hh, LRELU * hh)

    return pl.pallas_call(
        body,
        grid=grid,
        in_specs=[
            pl.BlockSpec((NC, R, D), lambda i: (0, i, 0)),
            pl.BlockSpec((NC, R, 1), lambda i: (0, i, 0)),
            pl.BlockSpec((R, D), lambda i: (i, 0)),
            pl.BlockSpec((D, D), lambda i: (0, 0)),
            pl.BlockSpec((1, D), lambda i: (0, 0)),
            pl.BlockSpec((D, D), lambda i: (0, 0)),
            pl.BlockSpec((1, D), lambda i: (0, 0)),
            pl.BlockSpec((1, D), lambda i: (0, 0)),
        ],
        out_specs=pl.BlockSpec((R, D), lambda i: (i, 0)),
        out_shape=jax.ShapeDtypeStruct((N, D), jnp.float32),
    )(p, deg3, h, Wl, bl.reshape(1, D), Wr, gamma.reshape(1, D),
      beta.reshape(1, D))


def kernel(x, edge_index, edge_weight, W1l, b1l, W1r, W2l, b2l, W2r,
           W3l, b3l, W3r, g1, be1, g2, be2):
    N, D = x.shape
    E = edge_weight.shape[0]
    src = edge_index[0]
    dst = edge_index[1]
    w128 = jnp.repeat(edge_weight, 16).reshape(E // 8, 128)
    z2 = jnp.zeros((N, D), jnp.float32)

    spmm_deg = _make_sc_spmm(N, D, E, with_deg=True)
    spmm = _make_sc_spmm(N, D, E, with_deg=False)

    p1, deg = spmm_deg(x, src, dst, w128, z2)
    deg3 = deg[:, :N].reshape(NC, N, 1)
    h1 = _dense_layer(p1, deg3, x, W1l, b1l, W1r, g1, be1, final=False)
    (p2,) = spmm(h1, src, dst, w128, z2)
    h2 = _dense_layer(p2, deg3, h1, W2l, b2l, W2r, g2, be2, final=False)
    (p3,) = spmm(h2, src, dst, w128, z2)
    out = _dense_layer(p3, deg3, h2, W3l, b3l, W3r, g1, be1, final=True)
    return out
